# baseline (speedup 1.0000x reference)
import jax
import jax.numpy as jnp
import numpy as np
from jax.experimental import pallas as pl
from jax.experimental.pallas import tpu as pltpu

_B = 128
_N = 100000
_W = 2048
_SPB = 1
_BW = _W * _SPB
_NCH = _N // _BW
_TW = _N - _NCH * _BW
_NBUF = 6

_TINY = np.float32(np.finfo(np.float32).tiny)
_ONE = np.float32(1.0)
_KEY1 = np.uint32(42)
_KS = (np.uint32(0), _KEY1, np.uint32(_KEY1 ^ np.uint32(0x1BD11BDA)))
_ROT = ((13, 15, 26, 6), (17, 29, 16, 24))
_IMAX = np.int32(np.iinfo(np.int32).max)


def _rotl(x, d):
    return (x << np.uint32(d)) | (x >> np.uint32(32 - d))


def _threefry_gumbel(base, goff):
    x1 = base + (goff + jnp.int32(_KEY1)).astype(jnp.uint32)
    x0 = x1
    x1 = x0 ^ _rotl(x1, _ROT[0][0])
    for r in _ROT[0][1:]:
        x0 = x0 + x1
        x1 = _rotl(x1, r)
        x1 = x0 ^ x1
    x0 = x0 + _KS[1]
    x1 = x1 + _KS[2] + np.uint32(1)
    for i in range(1, 5):
        for r in _ROT[i % 2]:
            x0 = x0 + x1
            x1 = _rotl(x1, r)
            x1 = x0 ^ x1
        x0 = x0 + _KS[(i + 1) % 3]
        x1 = x1 + _KS[(i + 2) % 3] + np.uint32(i + 1)
    bits = x0 ^ x1

    fb = (bits >> np.uint32(9)) | np.uint32(0x3F800000)
    f = jax.lax.bitcast_convert_type(fb, jnp.float32) - _ONE
    u = jnp.maximum(_TINY, f + _TINY)
    w = jnp.log(u) * np.float32(-1.0)
    return jnp.log(w)


def _gumbel_argmax_kernel(x_hbm, idx_ref, buf_ref, tbuf_ref, val_ref, arg_ref,
                          sems, tsem):
    def copy(c, slot):
        return pltpu.make_async_copy(
            x_hbm.at[:, pl.ds(c * _BW, _BW)], buf_ref.at[slot], sems.at[slot]
        )

    tail_copy = pltpu.make_async_copy(
        x_hbm.at[:, pl.ds(_NCH * _BW, _TW)], tbuf_ref, tsem
    )
    for i in range(_NBUF):
        copy(i, i).start()
    tail_copy.start()

    row = jax.lax.broadcasted_iota(jnp.uint32, (_B, _W), 0)
    cloc = jax.lax.broadcasted_iota(jnp.int32, (_B, _W), 1)
    base = row * np.uint32(_N) + cloc.astype(jnp.uint32)

    val_ref[...] = jnp.full((_B, 1), -jnp.inf, jnp.float32)
    arg_ref[...] = jnp.zeros((_B, 1), jnp.int32)

    def merge(m, cloc_arr, goff):
        cmax = jnp.max(m, axis=1, keepdims=True)
        cand = jnp.where(m == cmax, cloc_arr, _IMAX)
        carg = jnp.min(cand, axis=1, keepdims=True) + goff
        prev = val_ref[...]
        take = cmax > prev
        val_ref[...] = jnp.where(take, cmax, prev)
        arg_ref[...] = jnp.where(take, carg, arg_ref[...])

    def body(c, _):
        slot = jax.lax.rem(c, _NBUF)
        copy(c, slot).wait()
        for s in range(_SPB):
            goff = c * _BW + s * _W
            m = buf_ref[slot, :, pl.ds(s * _W, _W)] - _threefry_gumbel(base, goff)
            merge(m, cloc, goff)

        @pl.when(c + _NBUF < _NCH)
        def _():
            copy(c + _NBUF, jax.lax.rem(c + _NBUF, _NBUF)).start()

        return 0

    jax.lax.fori_loop(0, _NCH, body, 0)

    tail_copy.wait()
    trow = jax.lax.broadcasted_iota(jnp.uint32, (_B, _TW), 0)
    tcloc = jax.lax.broadcasted_iota(jnp.int32, (_B, _TW), 1)
    tbase = trow * np.uint32(_N) + tcloc.astype(jnp.uint32)
    tm = tbuf_ref[...] - _threefry_gumbel(tbase, _NCH * _BW)
    merge(tm, tcloc, _NCH * _BW)

    idx_ref[...] = arg_ref[...]


def kernel(logits):
    idx = pl.pallas_call(
        _gumbel_argmax_kernel,
        in_specs=[pl.BlockSpec(memory_space=pl.ANY)],
        out_specs=pl.BlockSpec(memory_space=pltpu.VMEM),
        out_shape=jax.ShapeDtypeStruct((_B, 1), jnp.int32),
        scratch_shapes=[
            pltpu.VMEM((_NBUF, _B, _BW), jnp.float32),
            pltpu.VMEM((_B, _TW), jnp.float32),
            pltpu.VMEM((_B, 1), jnp.float32),
            pltpu.VMEM((_B, 1), jnp.int32),
            pltpu.SemaphoreType.DMA((_NBUF,)),
            pltpu.SemaphoreType.DMA,
        ],
    )(logits)
    return idx.astype(jnp.int64)


# speedup vs baseline: 1.0134x; 1.0134x over previous
import functools

import jax
import jax.numpy as jnp
import numpy as np
from jax import lax
from jax.experimental import pallas as pl
from jax.experimental.pallas import tpu as pltpu
from jax.experimental.pallas import tpu_sc as plsc

_B = 128
_N = 100000

_SC_COLS = 0
_W = 2048
_SPB = 4
_BW = _W * _SPB
_NCH = _N // _BW
_TW = _N - _NCH * _BW
_TWS = (1696,)
_NBUF = 3

_TINY = np.float32(np.finfo(np.float32).tiny)
_ONE = np.float32(1.0)
_KEY1 = np.uint32(42)
_KS = (np.uint32(0), _KEY1, np.uint32(_KEY1 ^ np.uint32(0x1BD11BDA)))
_ROT = ((13, 15, 26, 6), (17, 29, 16, 24))
_IMAX = np.int32(np.iinfo(np.int32).max)

_LOGC = (1.0, -0.5, 0.33333302, -0.25000024, 0.20002577, -0.16668043,
         0.14212607, -0.12399221, 0.119239256, -0.117272295, 0.067402326)
_LN2_HI = np.float32(0.693359375)
_LN2_LO = np.float32(-2.12194440e-4)


def _rotl(x, d):
    return (x << np.uint32(d)) | (x >> np.uint32(32 - d))


def _threefry_u(x1):
    x0 = x1
    x1 = x0 ^ _rotl(x1, _ROT[0][0])
    for r in _ROT[0][1:]:
        x0 = x0 + x1
        x1 = _rotl(x1, r)
        x1 = x0 ^ x1
    x0 = x0 + _KS[1]
    x1 = x1 + _KS[2] + np.uint32(1)
    for i in range(1, 5):
        for r in _ROT[i % 2]:
            x0 = x0 + x1
            x1 = _rotl(x1, r)
            x1 = x0 ^ x1
        x0 = x0 + _KS[(i + 1) % 3]
        x1 = x1 + _KS[(i + 2) % 3] + np.uint32(i + 1)
    bits = x0 ^ x1

    fb = (bits >> np.uint32(9)) | np.uint32(0x3F800000)
    f = jax.lax.bitcast_convert_type(fb, jnp.float32) - _ONE
    return jnp.maximum(_TINY, f + _TINY)


def _tc_gumbel(base, goff):
    u = _threefry_u(base + (goff + jnp.int32(_KEY1)).astype(jnp.uint32))
    w = jnp.log(u) * np.float32(-1.0)
    return jnp.log(w)


def _tc_kernel(x_hbm, val_out, idx_out, buf_ref, tbuf_ref, val_ref, arg_ref,
               sems, tsem):
    def copy(c, slot):
        return pltpu.make_async_copy(
            x_hbm.at[:, pl.ds(_SC_COLS + c * _BW, _BW)],
            buf_ref.at[slot], sems.at[slot]
        )

    tail_copy = pltpu.make_async_copy(
        x_hbm.at[:, pl.ds(_SC_COLS + _NCH * _BW, _TW)], tbuf_ref, tsem
    )
    for i in range(_NBUF):
        copy(i, i).start()
    tail_copy.start()

    row = jax.lax.broadcasted_iota(jnp.uint32, (_B, _W), 0)
    cloc = jax.lax.broadcasted_iota(jnp.int32, (_B, _W), 1)
    base = row * np.uint32(_N) + cloc.astype(jnp.uint32)

    val_ref[...] = jnp.full((_B, 1), -jnp.inf, jnp.float32)
    arg_ref[...] = jnp.zeros((_B, 1), jnp.int32)

    def merge(m, cloc_arr, goff):
        cmax = jnp.max(m, axis=1, keepdims=True)
        cand = jnp.where(m == cmax, cloc_arr, _IMAX)
        carg = jnp.min(cand, axis=1, keepdims=True) + goff
        prev = val_ref[...]
        take = cmax > prev
        val_ref[...] = jnp.where(take, cmax, prev)
        arg_ref[...] = jnp.where(take, carg, arg_ref[...])

    def body(c, _):
        slot = jax.lax.rem(c, _NBUF)
        copy(c, slot).wait()
        for s in range(_SPB):
            goff = _SC_COLS + c * _BW + s * _W
            m = buf_ref[slot, :, pl.ds(s * _W, _W)] - _tc_gumbel(base, goff)
            merge(m, cloc, goff)

        @pl.when(c + _NBUF < _NCH)
        def _():
            copy(c + _NBUF, jax.lax.rem(c + _NBUF, _NBUF)).start()

        return 0

    jax.lax.fori_loop(0, _NCH, body, 0)

    tail_copy.wait()
    toff = 0
    for tw in _TWS:
        goff = _SC_COLS + _NCH * _BW + toff
        if tw == _W:
            tm = tbuf_ref[:, pl.ds(toff, tw)] - _tc_gumbel(base, goff)
            merge(tm, cloc, goff)
        else:
            trow = jax.lax.broadcasted_iota(jnp.uint32, (_B, tw), 0)
            tcloc = jax.lax.broadcasted_iota(jnp.int32, (_B, tw), 1)
            tbase = trow * np.uint32(_N) + tcloc.astype(jnp.uint32)
            tm = tbuf_ref[:, pl.ds(toff, tw)] - _tc_gumbel(tbase, goff)
            merge(tm, tcloc, goff)
        toff += tw

    val_out[...] = val_ref[...]
    idx_out[...] = arg_ref[...]


def _tc_call(logits):
    return pl.pallas_call(
        _tc_kernel,
        in_specs=[pl.BlockSpec(memory_space=pl.ANY)],
        out_specs=[
            pl.BlockSpec(memory_space=pltpu.VMEM),
            pl.BlockSpec(memory_space=pltpu.VMEM),
        ],
        out_shape=[
            jax.ShapeDtypeStruct((_B, 1), jnp.float32),
            jax.ShapeDtypeStruct((_B, 1), jnp.int32),
        ],
        scratch_shapes=[
            pltpu.VMEM((_NBUF, _B, _BW), jnp.float32),
            pltpu.VMEM((_B, _TW), jnp.float32),
            pltpu.VMEM((_B, 1), jnp.float32),
            pltpu.VMEM((_B, 1), jnp.int32),
            pltpu.SemaphoreType.DMA((_NBUF,)),
            pltpu.SemaphoreType.DMA,
        ],
    )(logits)


def kernel(logits):
    _, idx = _tc_call(logits)
    return idx.astype(jnp.int64)
